# baseline (speedup 1.0000x reference)
import functools

import jax
import jax.numpy as jnp
from jax import lax
from jax.experimental import pallas as pl
from jax.experimental.pallas import tpu as pltpu
from jax.experimental.pallas import tpu_sc as plsc

_NC = 2
_NS = 16
_NW = _NC * _NS
_L = 16

_EPS = 1e-5


def _table_body(arm_ref, color_ref, step_ref, gamma_ref, beta_ref, out_ref):
    na = arm_ref.shape[0]
    nc = color_ref.shape[0]
    ns = step_ref.shape[0]
    n = na * nc * ns

    def onehot(cols, idx_of_row):
        r = lax.broadcasted_iota(jnp.int32, (n, cols), 0)
        c = lax.broadcasted_iota(jnp.int32, (n, cols), 1)
        return (idx_of_row(r) == c).astype(jnp.float32)

    a_sel = onehot(na, lambda r: r // (nc * ns))
    c_sel = onehot(nc, lambda r: (r // ns) % nc)
    s_sel = onehot(ns, lambda r: r % ns)

    tok = (
        jnp.dot(a_sel, arm_ref[...], preferred_element_type=jnp.float32)
        + jnp.dot(c_sel, color_ref[...], preferred_element_type=jnp.float32)
        + jnp.dot(s_sel, step_ref[...], preferred_element_type=jnp.float32)
    )
    mean = jnp.mean(tok, axis=1, keepdims=True)
    d = tok - mean
    var = jnp.mean(d * d, axis=1, keepdims=True)
    row = d * lax.rsqrt(var + _EPS) * gamma_ref[...] + beta_ref[...]
    for r in range(out_ref.shape[0] // n):
        out_ref[pl.ds(r * n, n), :] = row


def _build_table(arm_w, color_w, step_w, gamma, beta, replicas):
    n = arm_w.shape[0] * color_w.shape[0] * step_w.shape[0]
    h = arm_w.shape[1]
    return pl.pallas_call(
        _table_body,
        out_shape=jax.ShapeDtypeStruct((replicas * n, h), jnp.float32),
    )(arm_w, color_w, step_w, gamma.reshape(1, h), beta.reshape(1, h))


def _make_gather(b, h, bpw, chunk, nrows):
    nchunk = bpw // chunk
    mesh = plsc.VectorSubcoreMesh(core_axis_name="c", subcore_axis_name="s")

    @functools.partial(
        pl.kernel,
        out_type=jax.ShapeDtypeStruct((b, 1, h), jnp.float32),
        mesh=mesh,
        scratch_types=[
            pltpu.VMEM((bpw,), jnp.int32),
            pltpu.VMEM((bpw,), jnp.int32),
            pltpu.VMEM((bpw,), jnp.int32),
            pltpu.VMEM((bpw,), jnp.int32),
            pltpu.VMEM((nrows, h), jnp.float32),
            pltpu.VMEM((2, chunk, h), jnp.float32),
            pltpu.SemaphoreType.DMA,
            pltpu.SemaphoreType.DMA,
        ],
        compiler_params=pltpu.CompilerParams(needs_layout_passes=False),
    )
    def gather_kernel(table_hbm, aidx_hbm, cidx_hbm, sidx_hbm, out_hbm,
                      aidx_v, cidx_v, sidx_v, fidx_v, table_v, rows_v,
                      tsem, ssem):
        wid = lax.axis_index("s") * _NC + lax.axis_index("c")
        base = wid * bpw

        tcopy = pltpu.async_copy(
            table_hbm.at[pl.ds(wid * nrows, nrows)], table_v, tsem)
        pltpu.sync_copy(aidx_hbm.at[pl.ds(base, bpw)], aidx_v)
        pltpu.sync_copy(cidx_hbm.at[pl.ds(base, bpw)], cidx_v)
        pltpu.sync_copy(sidx_hbm.at[pl.ds(base, bpw)], sidx_v)

        @pl.loop(0, bpw // _L)
        def _(j):
            sl = pl.ds(j * _L, _L)
            fidx_v[sl] = aidx_v[sl] * 16 + cidx_v[sl] * 4 + sidx_v[sl]

        tcopy.wait()
        lanes = lax.broadcasted_iota(jnp.int32, (_L,), 0)

        def expand(g, buf):
            for r in range(chunk):
                bvec = jnp.full((_L,), g * chunk + r, jnp.int32)
                row = plsc.load_gather(fidx_v, [bvec])
                for c in range(h // _L):
                    vals = plsc.load_gather(
                        table_v, [row, lanes + c * _L])
                    rows_v[buf, r, pl.ds(c * _L, _L)] = vals

        def fire_store(g, buf):
            pltpu.async_copy(rows_v.at[buf],
                             out_hbm.at[pl.ds(base + g * chunk, chunk), 0],
                             ssem)

        def drain_store(buf):
            pltpu.make_async_copy(
                table_hbm.at[pl.ds(0, chunk)], rows_v.at[buf], ssem
            ).wait()

        for g in range(2):
            expand(g, g)
            fire_store(g, g)

        @pl.loop(2, nchunk, step=2)
        def _(g0):
            for j in range(2):
                g = g0 + j
                drain_store(j)
                expand(g, j)
                fire_store(g, j)

        drain_store(0)
        drain_store(1)

    return gather_kernel


def kernel(arm_idx, color_idx, step_idx, arm_w, color_w, step_w, gamma, beta):
    b = arm_idx.shape[0]
    h = arm_w.shape[1]
    nrows = arm_w.shape[0] * color_w.shape[0] * step_w.shape[0]
    table = _build_table(arm_w, color_w, step_w, gamma, beta, replicas=_NW)

    bpw = b // _NW
    chunk = 4
    gather = _make_gather(b, h, bpw, chunk, nrows=nrows)
    return gather(
        table,
        arm_idx.astype(jnp.int32),
        color_idx.astype(jnp.int32),
        step_idx.astype(jnp.int32),
    )


# speedup vs baseline: 3.9471x; 3.9471x over previous
import functools

import jax
import jax.numpy as jnp
from jax import lax
from jax.experimental import pallas as pl
from jax.experimental.pallas import tpu as pltpu
from jax.experimental.pallas import tpu_sc as plsc

_NC = 2
_NS = 16
_NW = _NC * _NS
_L = 16

_EPS = 1e-5


def _table_body(arm_ref, color_ref, step_ref, gamma_ref, beta_ref, out_ref):
    na = arm_ref.shape[0]
    nc = color_ref.shape[0]
    ns = step_ref.shape[0]
    n = na * nc * ns

    def onehot(cols, idx_of_row):
        r = lax.broadcasted_iota(jnp.int32, (n, cols), 0)
        c = lax.broadcasted_iota(jnp.int32, (n, cols), 1)
        return (idx_of_row(r) == c).astype(jnp.float32)

    a_sel = onehot(na, lambda r: r // (nc * ns))
    c_sel = onehot(nc, lambda r: (r // ns) % nc)
    s_sel = onehot(ns, lambda r: r % ns)

    tok = (
        jnp.dot(a_sel, arm_ref[...], preferred_element_type=jnp.float32)
        + jnp.dot(c_sel, color_ref[...], preferred_element_type=jnp.float32)
        + jnp.dot(s_sel, step_ref[...], preferred_element_type=jnp.float32)
    )
    mean = jnp.mean(tok, axis=1, keepdims=True)
    d = tok - mean
    var = jnp.mean(d * d, axis=1, keepdims=True)
    row = d * lax.rsqrt(var + _EPS) * gamma_ref[...] + beta_ref[...]
    for r in range(out_ref.shape[0] // n):
        out_ref[pl.ds(r * n, n), :] = row


def _build_table(arm_w, color_w, step_w, gamma, beta, replicas):
    n = arm_w.shape[0] * color_w.shape[0] * step_w.shape[0]
    h = arm_w.shape[1]
    return pl.pallas_call(
        _table_body,
        out_shape=jax.ShapeDtypeStruct((replicas * n, h), jnp.float32),
    )(arm_w, color_w, step_w, gamma.reshape(1, h), beta.reshape(1, h))


def _make_gather(b, h, bpw, chunk, nbuf, nrows):
    nchunk = bpw // chunk
    mesh = plsc.VectorSubcoreMesh(core_axis_name="c", subcore_axis_name="s")

    @functools.partial(
        pl.kernel,
        out_type=jax.ShapeDtypeStruct((b, 1, h), jnp.float32),
        mesh=mesh,
        scratch_types=[
            pltpu.VMEM((bpw,), jnp.int32),
            pltpu.VMEM((bpw,), jnp.int32),
            pltpu.VMEM((bpw,), jnp.int32),
            pltpu.VMEM((bpw,), jnp.int32),
            pltpu.VMEM((nbuf, chunk, h), jnp.float32),
            pltpu.SemaphoreType.DMA,
            pltpu.SemaphoreType.DMA,
        ],
    )
    def gather_kernel(table_hbm, aidx_hbm, cidx_hbm, sidx_hbm, out_hbm,
                      aidx_v, cidx_v, sidx_v, fidx_v, rows_v, gsem, ssem):
        wid = lax.axis_index("s") * _NC + lax.axis_index("c")
        base = wid * bpw

        ca = pltpu.async_copy(aidx_hbm.at[pl.ds(base, bpw)], aidx_v, gsem)
        cc = pltpu.async_copy(cidx_hbm.at[pl.ds(base, bpw)], cidx_v, gsem)
        cs = pltpu.async_copy(sidx_hbm.at[pl.ds(base, bpw)], sidx_v, gsem)
        ca.wait(); cc.wait(); cs.wait()

        @pl.loop(0, bpw // _L)
        def _(j):
            sl = pl.ds(j * _L, _L)
            fidx_v[sl] = (aidx_v[sl] * 16 + cidx_v[sl] * 4 + sidx_v[sl]
                          + wid * nrows)

        def fire_gather(g, buf):
            idx = fidx_v[pl.ds(g * chunk, chunk)]
            pltpu.async_copy(table_hbm.at[idx], rows_v.at[buf], gsem)

        def drain(sem, buf):
            pltpu.make_async_copy(
                table_hbm.at[pl.ds(0, chunk)], rows_v.at[buf], sem
            ).wait()

        def fire_store(g, buf):
            pltpu.async_copy(rows_v.at[buf],
                             out_hbm.at[pl.ds(base + g * chunk, chunk), 0],
                             ssem)

        def step(g, gbuf):
            drain(gsem, gbuf)
            fire_store(g, gbuf)
            drain(ssem, gbuf)
            fire_gather(g + nbuf - 1, (g + nbuf - 1) % nbuf)

        for g in range(nbuf):
            fire_gather(g, g)
        drain(gsem, 0)
        fire_store(0, 0)

        main = (nchunk - nbuf - 1) // nbuf * nbuf
        @pl.loop(1, 1 + main, step=nbuf)
        def _(g0):
            for j in range(nbuf):
                g = g0 + j
                step(g, (1 + j) % nbuf)

        for g in range(1 + main, nchunk):
            if g + nbuf - 1 < nchunk:
                step(g, g % nbuf)
            else:
                drain(gsem, g % nbuf)
                fire_store(g, g % nbuf)
                drain(ssem, g % nbuf)
        drain(ssem, 0)

    return gather_kernel


def kernel(arm_idx, color_idx, step_idx, arm_w, color_w, step_w, gamma, beta):
    b = arm_idx.shape[0]
    h = arm_w.shape[1]
    nrows = arm_w.shape[0] * color_w.shape[0] * step_w.shape[0]
    table = _build_table(arm_w, color_w, step_w, gamma, beta, replicas=_NW)

    bpw = b // _NW
    chunk = 16
    gather = _make_gather(b, h, bpw, chunk, nbuf=3, nrows=nrows)
    return gather(
        table,
        arm_idx.astype(jnp.int32),
        color_idx.astype(jnp.int32),
        step_idx.astype(jnp.int32),
    )


# speedup vs baseline: 3.9719x; 1.0063x over previous
import functools

import jax
import jax.numpy as jnp
from jax import lax
from jax.experimental import pallas as pl
from jax.experimental.pallas import tpu as pltpu
from jax.experimental.pallas import tpu_sc as plsc

_NC = 2
_NS = 16
_NW = _NC * _NS
_L = 16

_EPS = 1e-5


def _table_body(arm_ref, color_ref, step_ref, gamma_ref, beta_ref, out_ref):
    na = arm_ref.shape[0]
    nc = color_ref.shape[0]
    ns = step_ref.shape[0]
    n = na * nc * ns

    def onehot(cols, idx_of_row):
        r = lax.broadcasted_iota(jnp.int32, (n, cols), 0)
        c = lax.broadcasted_iota(jnp.int32, (n, cols), 1)
        return (idx_of_row(r) == c).astype(jnp.float32)

    a_sel = onehot(na, lambda r: r // (nc * ns))
    c_sel = onehot(nc, lambda r: (r // ns) % nc)
    s_sel = onehot(ns, lambda r: r % ns)

    tok = (
        jnp.dot(a_sel, arm_ref[...], preferred_element_type=jnp.float32)
        + jnp.dot(c_sel, color_ref[...], preferred_element_type=jnp.float32)
        + jnp.dot(s_sel, step_ref[...], preferred_element_type=jnp.float32)
    )
    mean = jnp.mean(tok, axis=1, keepdims=True)
    d = tok - mean
    var = jnp.mean(d * d, axis=1, keepdims=True)
    row = d * lax.rsqrt(var + _EPS) * gamma_ref[...] + beta_ref[...]
    for r in range(out_ref.shape[0] // n):
        out_ref[pl.ds(r * n, n), :] = row


def _build_table(arm_w, color_w, step_w, gamma, beta, replicas):
    n = arm_w.shape[0] * color_w.shape[0] * step_w.shape[0]
    h = arm_w.shape[1]
    return pl.pallas_call(
        _table_body,
        out_shape=jax.ShapeDtypeStruct((replicas * n, h), jnp.float32),
    )(arm_w, color_w, step_w, gamma.reshape(1, h), beta.reshape(1, h))


def _make_gather(b, h, bpw, chunk, nbuf, nrows):
    nchunk = bpw // chunk
    mesh = plsc.VectorSubcoreMesh(core_axis_name="c", subcore_axis_name="s")

    @functools.partial(
        pl.kernel,
        out_type=jax.ShapeDtypeStruct((b, 1, h), jnp.float32),
        mesh=mesh,
        scratch_types=[
            pltpu.VMEM((bpw,), jnp.int32),
            pltpu.VMEM((bpw,), jnp.int32),
            pltpu.VMEM((bpw,), jnp.int32),
            pltpu.VMEM((bpw,), jnp.int32),
            pltpu.VMEM((nbuf, chunk, h), jnp.float32),
            pltpu.SemaphoreType.DMA,
            pltpu.SemaphoreType.DMA,
        ],
    )
    def gather_kernel(table_hbm, aidx_hbm, cidx_hbm, sidx_hbm, out_hbm,
                      aidx_v, cidx_v, sidx_v, fidx_v, rows_v, gsem, ssem):
        wid = lax.axis_index("s") * _NC + lax.axis_index("c")
        base = wid * bpw

        ca = pltpu.async_copy(aidx_hbm.at[pl.ds(base, bpw)], aidx_v, gsem)
        cc = pltpu.async_copy(cidx_hbm.at[pl.ds(base, bpw)], cidx_v, gsem)
        cs = pltpu.async_copy(sidx_hbm.at[pl.ds(base, bpw)], sidx_v, gsem)
        ca.wait(); cc.wait(); cs.wait()

        @pl.loop(0, bpw // _L)
        def _(j):
            sl = pl.ds(j * _L, _L)
            fidx_v[sl] = (aidx_v[sl] * 16 + cidx_v[sl] * 4 + sidx_v[sl]
                          + (wid // 2) * nrows)

        def fire_gather(g, buf):
            idx = fidx_v[pl.ds(g * chunk, chunk)]
            pltpu.async_copy(table_hbm.at[idx], rows_v.at[buf], gsem)

        def drain(sem, buf):
            pltpu.make_async_copy(
                table_hbm.at[pl.ds(0, chunk)], rows_v.at[buf], sem
            ).wait()

        def fire_store(g, buf):
            pltpu.async_copy(rows_v.at[buf],
                             out_hbm.at[pl.ds(base + g * chunk, chunk), 0],
                             ssem)

        def step(g, gbuf):
            drain(gsem, gbuf)
            fire_store(g, gbuf)
            drain(ssem, gbuf)
            fire_gather(g + nbuf - 1, (g + nbuf - 1) % nbuf)

        for g in range(nbuf):
            fire_gather(g, g)
        drain(gsem, 0)
        fire_store(0, 0)

        main = (nchunk - nbuf - 1) // nbuf * nbuf
        @pl.loop(1, 1 + main, step=nbuf)
        def _(g0):
            for j in range(nbuf):
                g = g0 + j
                step(g, (1 + j) % nbuf)

        for g in range(1 + main, nchunk):
            if g + nbuf - 1 < nchunk:
                step(g, g % nbuf)
            else:
                drain(gsem, g % nbuf)
                fire_store(g, g % nbuf)
                drain(ssem, g % nbuf)
        drain(ssem, 0)

    return gather_kernel


def kernel(arm_idx, color_idx, step_idx, arm_w, color_w, step_w, gamma, beta):
    b = arm_idx.shape[0]
    h = arm_w.shape[1]
    nrows = arm_w.shape[0] * color_w.shape[0] * step_w.shape[0]
    table = _build_table(arm_w, color_w, step_w, gamma, beta, replicas=_NW // 2)

    bpw = b // _NW
    chunk = 16
    gather = _make_gather(b, h, bpw, chunk, nbuf=3, nrows=nrows)
    return gather(
        table,
        arm_idx.astype(jnp.int32),
        color_idx.astype(jnp.int32),
        step_idx.astype(jnp.int32),
    )
